# baseline (speedup 1.0000x reference)
import functools

import jax
import jax.numpy as jnp
from jax import lax
from jax.experimental import pallas as pl
from jax.experimental.pallas import tpu as pltpu
from jax.experimental.pallas import tpu_sc as plsc

EMB = 64
BATCH = 16384
SEQ = 50

NC = 2
NS = 16
NW = NC * NS
DROWS_PER_W = BATCH // NW

G = 16
NCH = DROWS_PER_W // G
NB = 2

_mesh = plsc.VectorSubcoreMesh(core_axis_name="c", subcore_axis_name="s")


@functools.partial(
    pl.kernel,
    mesh=_mesh,
    out_type=jax.ShapeDtypeStruct((BATCH, SEQ, EMB), jnp.float32),
    scratch_types=[
        pltpu.VMEM((DROWS_PER_W, SEQ), jnp.int32),
        pltpu.VMEM((NB, G, SEQ, EMB), jnp.float32),
        pltpu.SemaphoreType.DMA,
        pltpu.SemaphoreType.DMA,
        pltpu.SemaphoreType.DMA,
    ],
    compiler_params=pltpu.CompilerParams(use_tc_tiling_on_sc=False),
)
def _gather_kernel(data_hbm, table_hbm, out_hbm, idx_v, rows_v, gsem, os0, os1):
    wid = lax.axis_index("s") * NC + lax.axis_index("c")
    row_base = wid * DROWS_PER_W
    osems = (os0, os1)

    pltpu.sync_copy(data_hbm.at[pl.ds(row_base, DROWS_PER_W)], idx_v)

    def body(h, carry):
        for b in range(NB):
            c = h * NB + b
            buf = rows_v.at[b]
            out_slc = out_hbm.at[pl.ds(row_base + c * G, G)]

            @pl.when(h > 0)
            def _():
                pltpu.make_async_copy(buf, out_slc, osems[b]).wait()

            copies = [
                pltpu.async_copy(
                    table_hbm.at[idx_v.at[c * G + j]],
                    buf.at[j],
                    gsem,
                )
                for j in range(G)
            ]
            for cp in copies:
                cp.wait()

            pltpu.async_copy(buf, out_slc, osems[b])
        return carry

    lax.fori_loop(0, NCH // NB, body, 0)

    for b in range(NB):
        pltpu.make_async_copy(
            rows_v.at[b], out_hbm.at[pl.ds(row_base, G)], osems[b]
        ).wait()


def kernel(data, W_i):
    return _gather_kernel(data, W_i)


# speedup vs baseline: 1.3181x; 1.3181x over previous
import functools

import jax
import jax.numpy as jnp
from jax import lax
from jax.experimental import pallas as pl
from jax.experimental.pallas import tpu as pltpu
from jax.experimental.pallas import tpu_sc as plsc

EMB = 64
BATCH = 16384
SEQ = 50
FLAT = BATCH * SEQ * EMB

NC = 2
NS = 16
NW = NC * NS
DROWS_PER_W = BATCH // NW
CHUNK_F = SEQ * EMB

G = 16
NCH = DROWS_PER_W // G
NB = 2

_mesh = plsc.VectorSubcoreMesh(core_axis_name="c", subcore_axis_name="s")


@functools.partial(
    pl.kernel,
    mesh=_mesh,
    out_type=jax.ShapeDtypeStruct((BATCH * SEQ, EMB), jnp.float32),
    scratch_types=[
        pltpu.VMEM((DROWS_PER_W, SEQ), jnp.int32),
        pltpu.VMEM((NB, G * SEQ, EMB), jnp.float32),
        pltpu.SemaphoreType.DMA,
        pltpu.SemaphoreType.DMA,
        pltpu.SemaphoreType.DMA,
    ],
    compiler_params=pltpu.CompilerParams(use_tc_tiling_on_sc=False),
)
def _gather_kernel(data_hbm, table_hbm, out_hbm, idx_v, rows_v, gsem, os0, os1):
    wid = lax.axis_index("s") * NC + lax.axis_index("c")
    row_base = wid * DROWS_PER_W
    osems = (os0, os1)

    pltpu.sync_copy(data_hbm.at[pl.ds(row_base, DROWS_PER_W)], idx_v)

    def body(h, carry):
        for b in range(NB):
            c = h * NB + b
            buf = rows_v.at[b]
            out_slc = out_hbm.at[pl.ds((row_base + c * G) * SEQ, G * SEQ)]
            bufw = buf

            @pl.when(h > 0)
            def _():
                pltpu.make_async_copy(bufw, out_slc, osems[b]).wait()

            copies = [
                pltpu.async_copy(
                    table_hbm.at[idx_v.at[c * G + j]],
                    buf.at[pl.ds(j * SEQ, SEQ)],
                    gsem,
                )
                for j in range(G)
            ]
            for cp in copies:
                cp.wait()

            pltpu.async_copy(bufw, out_slc, osems[b])
        return carry

    lax.fori_loop(0, NCH // NB, body, 0)

    for b in range(NB):
        pltpu.make_async_copy(
            rows_v.at[b],
            out_hbm.at[pl.ds(row_base * SEQ, G * SEQ)],
            osems[b],
        ).wait()


BB = 128


def _transpose_body(y_ref, z_ref):
    x = y_ref[...].reshape(BB, SEQ * EMB)
    z_ref[...] = x.T


_transpose_kernel = pl.pallas_call(
    _transpose_body,
    grid=(BATCH // BB,),
    in_specs=[
        pl.BlockSpec((BB * CHUNK_F // 128, 128), lambda i: (i, 0)),
    ],
    out_specs=pl.BlockSpec((SEQ * EMB, BB), lambda i: (0, i)),
    out_shape=jax.ShapeDtypeStruct((SEQ * EMB, BATCH), jnp.float32),
)


def kernel(data, W_i):
    flat = _gather_kernel(data, W_i)
    z = _transpose_kernel(flat.reshape(FLAT // 128, 128))
    return z.reshape(SEQ, EMB, BATCH).transpose(2, 0, 1)


# speedup vs baseline: 1.4821x; 1.1245x over previous
import functools

import jax
import jax.numpy as jnp
from jax import lax
from jax.experimental import pallas as pl
from jax.experimental.pallas import tpu as pltpu
from jax.experimental.pallas import tpu_sc as plsc

EMB = 64
BATCH = 16384
SEQ = 50
FLAT = BATCH * SEQ * EMB

NC = 2
NS = 16
NW = NC * NS
DROWS_PER_W = BATCH // NW
CHUNK_F = SEQ * EMB

G = 16
NCH = DROWS_PER_W // G
NB = 2

_mesh = plsc.VectorSubcoreMesh(core_axis_name="c", subcore_axis_name="s")


@functools.partial(
    pl.kernel,
    mesh=_mesh,
    out_type=jax.ShapeDtypeStruct((BATCH * SEQ, EMB), jnp.float32),
    scratch_types=[
        pltpu.VMEM((DROWS_PER_W, SEQ), jnp.int32),
        pltpu.VMEM((NB, G * SEQ, EMB), jnp.float32),
        pltpu.SemaphoreType.DMA,
        pltpu.SemaphoreType.DMA,
        pltpu.SemaphoreType.DMA,
    ],
    compiler_params=pltpu.CompilerParams(use_tc_tiling_on_sc=False),
)
def _gather_kernel(data_hbm, table_hbm, out_hbm, idx_v, rows_v, gsem, os0, os1):
    wid = lax.axis_index("s") * NC + lax.axis_index("c")
    row_base = wid * DROWS_PER_W
    osems = (os0, os1)

    pltpu.sync_copy(data_hbm.at[pl.ds(row_base, DROWS_PER_W)], idx_v)

    def body(h, carry):
        for b in range(NB):
            c = h * NB + b
            buf = rows_v.at[b]
            out_slc = out_hbm.at[pl.ds((row_base + c * G) * SEQ, G * SEQ)]
            bufw = buf

            @pl.when(h > 0)
            def _():
                pltpu.make_async_copy(bufw, out_slc, osems[b]).wait()

            copies = [
                pltpu.async_copy(
                    table_hbm.at[idx_v.at[c * G + j]],
                    buf.at[pl.ds(j * SEQ, SEQ)],
                    gsem,
                )
                for j in range(G)
            ]
            for cp in copies:
                cp.wait()

            pltpu.async_copy(bufw, out_slc, osems[b])
        return carry

    lax.fori_loop(0, NCH // NB, body, 0)

    for b in range(NB):
        pltpu.make_async_copy(
            rows_v.at[b],
            out_hbm.at[pl.ds(row_base * SEQ, G * SEQ)],
            osems[b],
        ).wait()


BB = 256


def _transpose_body(y_ref, z_ref):
    x = y_ref[...].reshape(BB, SEQ * EMB)
    z_ref[...] = x.T


_transpose_kernel = pl.pallas_call(
    _transpose_body,
    grid=(BATCH // BB,),
    in_specs=[
        pl.BlockSpec((BB * CHUNK_F // 128, 128), lambda i: (i, 0)),
    ],
    out_specs=pl.BlockSpec((SEQ * EMB, BB), lambda i: (0, i)),
    out_shape=jax.ShapeDtypeStruct((SEQ * EMB, BATCH), jnp.float32),
)


def kernel(data, W_i):
    flat = _gather_kernel(data, W_i)
    z = _transpose_kernel(flat.reshape(FLAT // 128, 128))
    return z.reshape(SEQ, EMB, BATCH).transpose(2, 0, 1)
